# baseline (speedup 1.0000x reference)
import functools
import math

import jax
import jax.numpy as jnp
from jax import lax
from jax.experimental import pallas as pl
from jax.experimental.pallas import tpu as pltpu
from jax.experimental.pallas import tpu_sc as plsc

D_MODEL = 64
SCALE = 8.0
LANES = 16
CT = 128
NB = 3


def _make_sc_gather(B, V):
    info = plsc.get_sparse_core_info()
    NC, NS = info.num_cores, info.num_subcores
    NW = NC * NS
    per_w = B // NW
    nch = per_w // CT
    tg = CT // LANES

    mesh = plsc.VectorSubcoreMesh(core_axis_name="c", subcore_axis_name="s")

    @functools.partial(
        pl.kernel,
        out_type=jax.ShapeDtypeStruct((B, D_MODEL), jnp.float32),
        mesh=mesh,
        scratch_types=[
            pltpu.VMEM((nch, CT), jnp.int32),
            [pltpu.VMEM((CT,), jnp.int32) for _ in range(NB)],
            [pltpu.VMEM((CT,), jnp.int32) for _ in range(NB)],
            [pltpu.VMEM((CT, 2 * D_MODEL), jnp.float32) for _ in range(NB)],
            [pltpu.VMEM((CT, D_MODEL), jnp.float32) for _ in range(NB)],
            [pltpu.SemaphoreType.DMA for _ in range(NB)],
            [pltpu.SemaphoreType.DMA for _ in range(NB)],
        ],
        compiler_params=pltpu.CompilerParams(needs_layout_passes=False),
    )
    def body(tab_hbm, idx_hbm, out_hbm, idx_all, pvs, hvs, bufs, obufs,
             gsems, ssems):
        wid = lax.axis_index("s") * NC + lax.axis_index("c")
        wrow = wid * per_w

        pltpu.sync_copy(idx_hbm.at[pl.ds(wid * nch, nch)], idx_all)

        def gather_desc(s):
            return pltpu.make_async_copy(tab_hbm.at[pvs[s]], bufs[s], gsems[s])

        def scatter_desc(c, s):
            return pltpu.make_async_copy(
                obufs[s], out_hbm.at[pl.ds(wrow + c * CT, CT)], ssems[s]
            )

        def pre(c, s):
            @pl.when(c >= NB)
            def _():
                scatter_desc(c - NB, s).wait()

            for g in range(tg):
                sl = pl.ds(g * LANES, LANES)
                v = idx_all[c, sl]
                pvs[s][sl] = v >> 1
                hvs[s][sl] = (v & 1) << 6
            gather_desc(s).start()

        def post(c, s):
            gather_desc(s).wait()
            buf, obuf, hv_ref = bufs[s], obufs[s], hvs[s]
            ci = lax.iota(jnp.int32, LANES)

            def tok_group(g, carry):
                hv = hv_ref[pl.ds(g * LANES, LANES)]
                for l in range(LANES):
                    lv = jnp.full((LANES,), l, jnp.int32)
                    hb = lax.gather(
                        hv, lv[:, None],
                        lax.GatherDimensionNumbers(
                            offset_dims=(), collapsed_slice_dims=(0,),
                            start_index_map=(0,)),
                        (1,), mode=lax.GatherScatterMode.PROMISE_IN_BOUNDS,
                    )
                    t = g * LANES + l
                    rv = jnp.full((LANES,), t, jnp.int32)
                    for j in range(D_MODEL // LANES):
                        colv = hb + (j * LANES) + ci
                        v = plsc.load_gather(buf, [rv, colv])
                        obuf[t, pl.ds(j * LANES, LANES)] = v * SCALE
                return carry

            lax.fori_loop(0, tg, tok_group, 0)
            scatter_desc(c, s).start()

        def step(c, s_pre, s_post):
            @pl.when(c < nch)
            def _():
                pre(c, s_pre)

            c2 = c - 2

            @pl.when(jnp.logical_and(c2 >= 0, c2 < nch))
            def _():
                post(c2, s_post)

        n_steps = nch + 2
        n_rounds = (n_steps + NB - 1) // NB

        def round_body(k, carry):
            for t in range(NB):
                step(k * NB + t, t, (t + 1) % NB)
            return carry

        lax.fori_loop(0, n_rounds, round_body, 0)

        for c in range(nch - NB, nch):
            scatter_desc(c, c % NB).wait()

    return body


def kernel(token_ids, embedding_weights):
    BATCH, HIST = token_ids.shape
    B = BATCH * HIST
    V = embedding_weights.shape[0]
    tab2 = embedding_weights.reshape(V // 2, 2 * D_MODEL)
    idx = token_ids.reshape(B // CT, CT)
    out = _make_sc_gather(B, V)(tab2, idx)
    return out.reshape(BATCH, HIST, D_MODEL)


# speedup vs baseline: 1.1329x; 1.1329x over previous
import functools
import math

import jax
import jax.numpy as jnp
from jax import lax
from jax.experimental import pallas as pl
from jax.experimental.pallas import tpu as pltpu
from jax.experimental.pallas import tpu_sc as plsc

D_MODEL = 64
SCALE = 8.0
LANES = 16
G = 128
S = 2
CH = G * S
NB = 4


def _make_sc_gather(B, V):
    info = plsc.get_sparse_core_info()
    NC, NS = info.num_cores, info.num_subcores
    NW = NC * NS
    per_w = B // NW
    n_idx = per_w // G
    nch = per_w // CH

    mesh = plsc.VectorSubcoreMesh(core_axis_name="c", subcore_axis_name="s")

    @functools.partial(
        pl.kernel,
        out_type=jax.ShapeDtypeStruct((B, D_MODEL), jnp.float32),
        mesh=mesh,
        scratch_types=[
            pltpu.VMEM((n_idx, G), jnp.int32),
            [pltpu.VMEM((CH, D_MODEL), jnp.float32) for _ in range(NB)],
            [pltpu.SemaphoreType.DMA for _ in range(NB)],
            [pltpu.SemaphoreType.DMA for _ in range(NB)],
        ],
        compiler_params=pltpu.CompilerParams(use_tc_tiling_on_sc=False),
    )
    def body(table_hbm, idx_hbm, out_hbm, idx_all, bufs, gsems, ssems):
        wid = lax.axis_index("s") * NC + lax.axis_index("c")
        wrow = wid * per_w

        pltpu.sync_copy(idx_hbm.at[pl.ds(wid * n_idx, n_idx)], idx_all)

        def gather_descs(c, b):
            return [
                pltpu.make_async_copy(
                    table_hbm.at[idx_all.at[c * S + j]],
                    bufs[b].at[pl.ds(j * G, G)],
                    gsems[b],
                )
                for j in range(S)
            ]

        def scatter_desc(c, b):
            return pltpu.make_async_copy(
                bufs[b], out_hbm.at[pl.ds(wrow + c * CH, CH)], ssems[b]
            )

        def pre(c, b, waits_scatter):
            if waits_scatter:
                scatter_desc(c - NB, b).wait()
            for d in gather_descs(c, b):
                d.start()

        def post(c, b):
            for d in gather_descs(c, b):
                d.wait()
            buf = bufs[b]

            @plsc.parallel_loop(0, CH, unroll=4)
            def _scale(i):
                for j in range(D_MODEL // LANES):
                    sl = pl.ds(j * LANES, LANES)
                    buf[i, sl] = buf[i, sl] * SCALE

            scatter_desc(c, b).start()

        pre(0, 0, False)
        pre(1, 1, False)
        pre(2, 2, False)
        post(0, 0)
        pre(3, 3, False)
        post(1, 1)

        def round_body(k, carry):
            c0 = k * NB
            for b in range(NB):
                pre(c0 + b, b, True)
                post(c0 + b - 2, (b - 2) % NB)
            return carry

        lax.fori_loop(1, nch // NB, round_body, 0)

        post(nch - 2, (nch - 2) % NB)
        post(nch - 1, (nch - 1) % NB)
        for b in range(NB):
            scatter_desc(nch - NB + b, (nch - NB + b) % NB).wait()

    return body


def kernel(token_ids, embedding_weights):
    BATCH, HIST = token_ids.shape
    B = BATCH * HIST
    V = embedding_weights.shape[0]
    idx = token_ids.reshape(B // G, G)
    out = _make_sc_gather(B, V)(embedding_weights, idx)
    return out.reshape(BATCH, HIST, D_MODEL)


# speedup vs baseline: 1.5939x; 1.4069x over previous
import functools
import math

import jax
import jax.numpy as jnp
from jax import lax
from jax.experimental import pallas as pl
from jax.experimental.pallas import tpu as pltpu
from jax.experimental.pallas import tpu_sc as plsc

D_MODEL = 64
SCALE = 8.0
LANES = 16
CT = 128
NB = 4


def _make_sc_gather(BATCH, HIST, V):
    info = plsc.get_sparse_core_info()
    NC, NS = info.num_cores, info.num_subcores
    NW = NC * NS
    assert BATCH % (CT * NW) == 0
    nch = HIST
    tg = CT // LANES

    mesh = plsc.VectorSubcoreMesh(core_axis_name="c", subcore_axis_name="s")

    @functools.partial(
        pl.kernel,
        out_type=jax.ShapeDtypeStruct((HIST, D_MODEL, BATCH), jnp.float32),
        mesh=mesh,
        scratch_types=[
            pltpu.VMEM((nch, CT), jnp.int32),
            [pltpu.VMEM((CT,), jnp.int32) for _ in range(NB)],
            [pltpu.VMEM((CT,), jnp.int32) for _ in range(NB)],
            [pltpu.VMEM((CT, 2 * D_MODEL), jnp.float32) for _ in range(NB)],
            [pltpu.VMEM((D_MODEL, CT), jnp.float32) for _ in range(NB)],
            [pltpu.SemaphoreType.DMA for _ in range(NB)],
            [pltpu.SemaphoreType.DMA for _ in range(NB)],
        ],
        compiler_params=pltpu.CompilerParams(needs_layout_passes=False),
    )
    def body(tab_hbm, idx_hbm, out_hbm, idx_all, pvs, hvs, bufs, tbufs,
             gsems, ssems):
        wid = lax.axis_index("s") * NC + lax.axis_index("c")
        bcol = wid * CT

        for th in range(nch // 8):
            pltpu.sync_copy(
                idx_hbm.at[pl.ds(th * 8, 8), pl.ds(bcol, CT)],
                idx_all.at[pl.ds(th * 8, 8)],
            )

        def gather_desc(s):
            return pltpu.make_async_copy(tab_hbm.at[pvs[s]], bufs[s], gsems[s])

        def scatter_descs(h, s):
            return [
                pltpu.make_async_copy(
                    tbufs[s].at[pl.ds(tc * 8, 8)],
                    out_hbm.at[h, pl.ds(tc * 8, 8), pl.ds(bcol, CT)],
                    ssems[s],
                )
                for tc in range(D_MODEL // 8)
            ]

        def pre(h, s):
            @pl.when(h >= NB)
            def _():
                for d in scatter_descs(h - NB, s):
                    d.wait()

            for g in range(tg):
                sl = pl.ds(g * LANES, LANES)
                v = idx_all[h, sl]
                pvs[s][sl] = v >> 1
                hvs[s][sl] = (v & 1) << 6
            gather_desc(s).start()

        def post(h, s):
            gather_desc(s).wait()
            buf, tbuf, hv_ref = bufs[s], tbufs[s], hvs[s]
            ci = lax.iota(jnp.int32, LANES)
            rowvs = [lax.iota(jnp.int32, LANES) + (g * LANES) for g in range(tg)]
            hvv = [hv_ref[pl.ds(g * LANES, LANES)] for g in range(tg)]

            @plsc.parallel_loop(0, LANES, unroll=2)
            def _sel(k):
                w = (ci + jnp.full((LANES,), k, jnp.int32)) & (LANES - 1)
                for j in range(D_MODEL // LANES):
                    colv = w + (j * LANES)
                    for g in range(tg):
                        v = plsc.load_gather(buf, [rowvs[g], hvv[g] + colv])
                        plsc.store_scatter(tbuf, [colv, rowvs[g]], v * SCALE)

            for d in scatter_descs(h, s):
                d.start()

        def step(h, s_pre, s_post):
            @pl.when(h < nch)
            def _():
                pre(h, s_pre)

            h2 = h - 2

            @pl.when(jnp.logical_and(h2 >= 0, h2 < nch))
            def _():
                post(h2, s_post)

        n_steps = nch + 2
        n_rounds = (n_steps + NB - 1) // NB

        def round_body(k, carry):
            for t in range(NB):
                step(k * NB + t, t, (t + 2) % NB)
            return carry

        lax.fori_loop(0, n_rounds, round_body, 0)

        for h in range(nch - NB, nch):
            for d in scatter_descs(h, h % NB):
                d.wait()

    return body


def kernel(token_ids, embedding_weights):
    BATCH, HIST = token_ids.shape
    V = embedding_weights.shape[0]
    tab2 = embedding_weights.reshape(V // 2, 2 * D_MODEL)
    idxT = token_ids.T
    out3 = _make_sc_gather(BATCH, HIST, V)(tab2, idxT)
    return out3.transpose(2, 0, 1)
